# baseline (speedup 1.0000x reference)
import functools

import jax
import jax.numpy as jnp
from jax import lax
from jax.experimental import pallas as pl
from jax.experimental.pallas import tpu as pltpu
from jax.experimental.pallas import tpu_sc as plsc

_POOL = 100000
_D = 128
_B = 4096
_S = 26
_NW = 32
_BPW = _B // _NW
_NBUF = 6
_PRIME = _NBUF - 1


def _make_kernel():
    mesh = plsc.VectorSubcoreMesh(core_axis_name="c", subcore_axis_name="s")

    @functools.partial(
        pl.kernel,
        mesh=mesh,
        out_type=jax.ShapeDtypeStruct((_S, _B, _D), jnp.float32),
        scratch_types=(
            [pltpu.VMEM((_S, _BPW), jnp.int32),
             pltpu.VMEM((_NBUF, _BPW, _D), jnp.float32)]
            + [pltpu.SemaphoreType.DMA] * (2 * _NBUF)
        ),
    )
    def gather_kernel(idx_hbm, table_hbm, out_hbm, idx_v, bufs, *sems):
        gsems = sems[:_NBUF]
        wsems = sems[_NBUF:]
        wid = lax.axis_index("s") * 2 + lax.axis_index("c")
        wb = wid * _BPW
        pltpu.sync_copy(idx_hbm.at[wid], idx_v)

        def g_halves(s, b):
            h = _BPW // 2
            return [
                pltpu.make_async_copy(
                    table_hbm.at[idx_v.at[s, pl.ds(k * h, h)]],
                    bufs.at[b, pl.ds(k * h, h)], gsems[b])
                for k in range(2)
            ]

        def w_copy(s, b):
            return pltpu.make_async_copy(
                bufs.at[b], out_hbm.at[s, pl.ds(wb, _BPW)], wsems[b])

        for s in range(_PRIME):
            for d in g_halves(s, s):
                d.start()

        for s in range(_S):
            b = s % _NBUF
            for d in g_halves(s, b):
                d.wait()
            w_copy(s, b).start()
            ns = s + _PRIME
            if ns < _S:
                nb = ns % _NBUF
                if ns >= _NBUF:
                    w_copy(ns - _NBUF, nb).wait()
                for d in g_halves(ns, nb):
                    d.start()

        for s in range(_S - _NBUF, _S):
            w_copy(s, s % _NBUF).wait()

    return gather_kernel


_gather = _make_kernel()


def kernel(indices, embedding):
    idx = indices.astype(jnp.int32).T.reshape(_S, _NW, _BPW).transpose(1, 0, 2)
    out = _gather(idx, embedding)
    return out.transpose(1, 0, 2)


# speedup vs baseline: 1.0062x; 1.0062x over previous
import functools

import jax
import jax.numpy as jnp
from jax import lax
from jax.experimental import pallas as pl
from jax.experimental.pallas import tpu as pltpu
from jax.experimental.pallas import tpu_sc as plsc

_POOL = 100000
_D = 128
_B = 4096
_S = 26
_NW = 32
_BPW = _B // _NW
_NBUF = 7
_PRIME = _NBUF - 1


def _make_kernel():
    mesh = plsc.VectorSubcoreMesh(core_axis_name="c", subcore_axis_name="s")

    @functools.partial(
        pl.kernel,
        mesh=mesh,
        out_type=jax.ShapeDtypeStruct((_S, _B, _D), jnp.float32),
        scratch_types=(
            [pltpu.VMEM((_S, _BPW), jnp.int32),
             pltpu.VMEM((_NBUF, _BPW, _D), jnp.float32)]
            + [pltpu.SemaphoreType.DMA] * (2 * _NBUF)
        ),
    )
    def gather_kernel(idx_hbm, table_hbm, out_hbm, idx_v, bufs, *sems):
        gsems = sems[:_NBUF]
        wsems = sems[_NBUF:]
        wid = lax.axis_index("s") * 2 + lax.axis_index("c")
        wb = wid * _BPW
        pltpu.sync_copy(idx_hbm.at[wid], idx_v)

        def g_copy(s, b):
            return pltpu.make_async_copy(
                table_hbm.at[idx_v.at[s]], bufs.at[b], gsems[b])

        def w_copy(s, b):
            return pltpu.make_async_copy(
                bufs.at[b], out_hbm.at[s, pl.ds(wb, _BPW)], wsems[b])

        for s in range(_PRIME):
            g_copy(s, s).start()

        for s in range(_S):
            b = s % _NBUF
            g_copy(s, b).wait()
            w_copy(s, b).start()
            ns = s + _PRIME
            if ns < _S:
                nb = ns % _NBUF
                if ns >= _NBUF:
                    w_copy(ns - _NBUF, nb).wait()
                g_copy(ns, nb).start()

        for s in range(_S - _NBUF, _S):
            w_copy(s, s % _NBUF).wait()

    return gather_kernel


_gather = _make_kernel()


def kernel(indices, embedding):
    idx = indices.astype(jnp.int32).T.reshape(_S, _NW, _BPW).transpose(1, 0, 2)
    out = _gather(idx, embedding)
    return out.transpose(1, 0, 2)


# speedup vs baseline: 1.0158x; 1.0095x over previous
import functools

import jax
import jax.numpy as jnp
from jax import lax
from jax.experimental import pallas as pl
from jax.experimental.pallas import tpu as pltpu
from jax.experimental.pallas import tpu_sc as plsc

_POOL = 100000
_D = 128
_B = 4096
_S = 26
_NW = 32
_BPW = _B // _NW
_NBUF = 6
_PRIME = _NBUF - 1


def _make_kernel():
    mesh = plsc.VectorSubcoreMesh(core_axis_name="c", subcore_axis_name="s")

    @functools.partial(
        pl.kernel,
        mesh=mesh,
        out_type=jax.ShapeDtypeStruct((_S, _B, _D), jnp.float32),
        scratch_types=(
            [pltpu.VMEM((_S, _BPW), jnp.int32),
             pltpu.VMEM((_NBUF, _BPW, _D), jnp.float32)]
            + [pltpu.SemaphoreType.DMA] * (2 * _NBUF)
        ),
    )
    def gather_kernel(idx_hbm, table_hbm, out_hbm, idx_v, bufs, *sems):
        gsems = sems[:_NBUF]
        wsems = sems[_NBUF:]
        wid = lax.axis_index("s") * 2 + lax.axis_index("c")
        wb = wid * _BPW
        pltpu.sync_copy(idx_hbm.at[wid], idx_v)

        def g_copy(s, b):
            return pltpu.make_async_copy(
                table_hbm.at[idx_v.at[s]], bufs.at[b], gsems[b])

        def w_copy(s, b):
            return pltpu.make_async_copy(
                bufs.at[b], out_hbm.at[s, pl.ds(wb, _BPW)], wsems[b])

        for s in range(_PRIME):
            g_copy(s, s).start()

        for s in range(_S):
            b = s % _NBUF
            g_copy(s, b).wait()
            w_copy(s, b).start()
            ns = s + _PRIME
            if ns < _S:
                nb = ns % _NBUF
                if ns >= _NBUF:
                    w_copy(ns - _NBUF, nb).wait()
                g_copy(ns, nb).start()

        for s in range(_S - _NBUF, _S):
            w_copy(s, s % _NBUF).wait()

    return gather_kernel


_gather = _make_kernel()


def kernel(indices, embedding):
    idx = indices.astype(jnp.int32).T.reshape(_S, _NW, _BPW).transpose(1, 0, 2)
    out = _gather(idx, embedding)
    return out.transpose(1, 0, 2)
